# baseline (speedup 1.0000x reference)
import functools

import jax
import jax.numpy as jnp
from jax import lax
from jax.experimental import pallas as pl
from jax.experimental.pallas import tpu as pltpu
from jax.experimental.pallas import tpu_sc as plsc

N = 50000
E = 800000
B = 512
DF = 78
H = 300
FF = 1024
OUT = 128

NC = 2
NS = 16
LANES = 16

EDGE_SENTINEL = 1 << 29
EP = 819200
DEG_CHUNK = EP // (NC * NS)
SUB = 6400
NSUB_DEG = DEG_CHUNK // SUB
NVREG = SUB // LANES

SCAT_CHUNK = EP // NS
SSUB = 2048
NSUB_SCAT = SCAT_CHUNK // SSUB
SVREG = SSUB // LANES
CHUNK = 64
CSH = 6
KROWS = SSUB // CHUNK

NBLK = 14
NBS = 3840
NP = NBLK * NBS
BLK_PER_SC = NBLK // NC
STRIPE = NBS // NS
HP = 384
SC_TILING_TC = False

BM = 1000
NGRID = N // BM


def _sc_mesh():
  return plsc.VectorSubcoreMesh(
      core_axis_name="c", subcore_axis_name="s", num_cores=NC,
      num_subcores=NS)


def _deg_body(dst_hbm, zeros_hbm, deg_out, acc, dst_sub):
  c = lax.axis_index("c")
  s = lax.axis_index("s")
  wid = c * NS + s
  ones16 = jnp.ones((LANES,), jnp.float32)
  pltpu.sync_copy(zeros_hbm, acc)

  def sub_body(u, _):
    base = wid * DEG_CHUNK + u * SUB
    pltpu.sync_copy(dst_hbm.at[pl.ds(base, SUB)], dst_sub)

    def vec_body(i, _):
      d16 = dst_sub[pl.ds(i * LANES, LANES)]
      idx = jnp.minimum(d16, N)
      plsc.addupdate_scatter(acc, [idx], ones16)
      return 0

    lax.fori_loop(0, NVREG, vec_body, 0)
    return 0

  lax.fori_loop(0, NSUB_DEG, sub_body, 0)
  pltpu.sync_copy(acc, deg_out.at[wid])


def _deg_counts(dst_pad, zeros_n1):
  kfn = pl.kernel(
      _deg_body,
      out_type=jax.ShapeDtypeStruct((NC * NS, N + 1), jnp.float32),
      mesh=_sc_mesh(),
      scratch_types=[
          pltpu.VMEM((N + 1,), jnp.float32),
          pltpu.VMEM((SUB,), jnp.int32),
      ],
      compiler_params=pltpu.CompilerParams(needs_layout_passes=False),
  )
  return kfn(dst_pad, zeros_n1)


def _scatter_body(g_hbm, src_hbm, dst_hbm, zrows_hbm, s_out,
                  acc, src_sub, dst_sub, gsrc_idx, sdst_idx, rowbuf, sem):
  c = lax.axis_index("c")
  s = lax.axis_index("s")

  def blk_body(kb, _):
    blk = c * BLK_PER_SC + kb
    lo = blk * NBS
    hi = lo + NBS

    pltpu.sync_copy(zrows_hbm, acc.at[pl.ds(s * STRIPE, STRIPE)])
    plsc.subcore_barrier()

    def sub_body(u, _):
      base = s * SCAT_CHUNK + u * SSUB
      pltpu.sync_copy(src_hbm.at[pl.ds(base, SSUB)], src_sub)
      pltpu.sync_copy(dst_hbm.at[pl.ds(base, SSUB)], dst_sub)

      def compact(i, n):
        s16 = src_sub[pl.ds(i * LANES, LANES)]
        d16 = dst_sub[pl.ds(i * LANES, LANES)]
        m = (d16 >= lo) & (d16 < hi)
        inc = m.astype(jnp.int32)
        cs = plsc.cumsum(inc)
        pos = (cs - inc) + n
        row = lax.shift_right_logical(pos, CSH)
        col = lax.bitwise_and(pos, CHUNK - 1)
        plsc.store_scatter(gsrc_idx, [row, col], s16, mask=m)
        plsc.store_scatter(sdst_idx, [row, col], d16 - lo, mask=m)
        return n + jnp.sum(inc)

      n = lax.fori_loop(0, SVREG, compact, jnp.int32(0))
      nch = (n + CHUNK - 1) >> CSH

      iota16 = lax.iota(jnp.int32, LANES)
      zero16 = jnp.zeros((LANES,), jnp.int32)
      trash16 = jnp.full((LANES,), NBS, jnp.int32)

      def pad_body(t, _):
        p = n + t * LANES + iota16
        m = p < (nch << CSH)
        row = lax.shift_right_logical(p, CSH)
        col = lax.bitwise_and(p, CHUNK - 1)
        plsc.store_scatter(gsrc_idx, [row, col], zero16, mask=m)
        plsc.store_scatter(sdst_idx, [row, col], trash16, mask=m)
        return 0
      lax.fori_loop(0, CHUNK // LANES, pad_body, 0)

      def xfer(j, _):
        pltpu.async_copy(g_hbm.at[gsrc_idx.at[j]], rowbuf, sem).wait()
        pltpu.sync_copy(rowbuf, acc.at[sdst_idx.at[j]], add=True)
        return 0
      lax.fori_loop(0, nch, xfer, 0)
      return 0

    lax.fori_loop(0, NSUB_SCAT, sub_body, 0)
    plsc.subcore_barrier()

    r = s * STRIPE
    pltpu.sync_copy(acc.at[pl.ds(r, STRIPE)], s_out.at[pl.ds(lo + r, STRIPE)])
    plsc.subcore_barrier()
    return 0

  lax.fori_loop(0, BLK_PER_SC, blk_body, 0)


def _edge_scatter(g, src_pad, dst_pad, zrows):
  kfn = pl.kernel(
      _scatter_body,
      out_type=jax.ShapeDtypeStruct((NP, HP), jnp.float32),
      mesh=_sc_mesh(),
      scratch_types=[
          pltpu.VMEM_SHARED((NBS + 1, HP), jnp.float32),
          pltpu.VMEM((SSUB,), jnp.int32),
          pltpu.VMEM((SSUB,), jnp.int32),
          pltpu.VMEM((KROWS, CHUNK), jnp.int32),
          pltpu.VMEM((KROWS, CHUNK), jnp.int32),
          pltpu.VMEM((CHUNK, HP), jnp.float32),
          pltpu.SemaphoreType.DMA,
      ],
      compiler_params=pltpu.CompilerParams(
          needs_layout_passes=False, use_tc_tiling_on_sc=SC_TILING_TC),
  )
  return kfn(g, src_pad, dst_pad, zrows)


def _g1_body(x_ref, w1_ref, degp_ref, g1_ref, dinv_ref):
  deg = degp_ref[...]
  ones = jnp.ones((NC * NS, 1), jnp.float32)
  dsum = jnp.dot(deg, ones, preferred_element_type=jnp.float32)
  dinv = lax.rsqrt(dsum + 1.0)
  h = jnp.dot(x_ref[...], w1_ref[...], preferred_element_type=jnp.float32)
  g1_ref[...] = dinv * h
  dinv_ref[...] = dinv


def _g1_kernel(x, w1, degp):
  return pl.pallas_call(
      _g1_body,
      grid=(NGRID,),
      in_specs=[
          pl.BlockSpec((BM, DF), lambda i: (i, 0)),
          pl.BlockSpec((DF, HP), lambda i: (0, 0)),
          pl.BlockSpec((BM, NC * NS), lambda i: (i, 0)),
      ],
      out_specs=[
          pl.BlockSpec((BM, HP), lambda i: (i, 0)),
          pl.BlockSpec((BM, 1), lambda i: (i, 0)),
      ],
      out_shape=[
          jax.ShapeDtypeStruct((N, HP), jnp.float32),
          jax.ShapeDtypeStruct((N, 1), jnp.float32),
      ],
      compiler_params=pltpu.CompilerParams(
          dimension_semantics=("parallel",)),
  )(x, w1, degp)


def _g2_body(s1_ref, g1_ref, dinv_ref, b1_ref, w2_ref, g2_ref):
  dinv = dinv_ref[...]
  y1 = jnp.maximum(dinv * (s1_ref[...] + g1_ref[...]) + b1_ref[...], 0.0)
  h = jnp.dot(y1, w2_ref[...], preferred_element_type=jnp.float32)
  g2_ref[...] = dinv * h


def _g2_kernel(s1, g1, dinv, b1r, w2):
  return pl.pallas_call(
      _g2_body,
      grid=(NGRID,),
      in_specs=[
          pl.BlockSpec((BM, HP), lambda i: (i, 0)),
          pl.BlockSpec((BM, HP), lambda i: (i, 0)),
          pl.BlockSpec((BM, 1), lambda i: (i, 0)),
          pl.BlockSpec((1, HP), lambda i: (0, 0)),
          pl.BlockSpec((HP, HP), lambda i: (0, 0)),
      ],
      out_specs=pl.BlockSpec((BM, HP), lambda i: (i, 0)),
      out_shape=jax.ShapeDtypeStruct((N, HP), jnp.float32),
      compiler_params=pltpu.CompilerParams(
          dimension_semantics=("parallel",)),
  )(s1, g1, dinv, b1r, w2)


def _pool_body(s2_ref, g2_ref, dinv_ref, b2_ref, batch_ref,
               w3_ref, b3_ref, w4_ref, b4_ref, out_ref, pooled):
  i = pl.program_id(0)

  @pl.when(i == 0)
  def _():
    pooled[...] = jnp.full((B, HP), -1e30, jnp.float32)

  dinv = dinv_ref[...]
  y2 = jnp.maximum(dinv * (s2_ref[...] + g2_ref[...]) + b2_ref[...], 0.0)
  bvec = batch_ref[...]
  g_first = bvec[0, 0]
  g_last = bvec[BM - 1, 0]

  def seg_body(g, _):
    m = bvec == g
    part = jnp.max(jnp.where(m, y2, -1e30), axis=0, keepdims=True)
    cur = pooled[pl.ds(g, 1), :]
    pooled[pl.ds(g, 1), :] = jnp.maximum(cur, part)
    return 0

  lax.fori_loop(g_first, g_last + 1, seg_body, 0)

  @pl.when(i == NGRID - 1)
  def _():
    p = jnp.maximum(pooled[...], 0.0)
    h3 = jnp.maximum(
        jnp.dot(p, w3_ref[...], preferred_element_type=jnp.float32)
        + b3_ref[...], 0.0)
    o = jnp.maximum(
        jnp.dot(h3, w4_ref[...], preferred_element_type=jnp.float32)
        + b4_ref[...], 0.0)
    out_ref[...] = o


def _pool_kernel(s2, g2, dinv, b2r, batch2d, w3, b3r, w4, b4r):
  return pl.pallas_call(
      _pool_body,
      grid=(NGRID,),
      in_specs=[
          pl.BlockSpec((BM, HP), lambda i: (i, 0)),
          pl.BlockSpec((BM, HP), lambda i: (i, 0)),
          pl.BlockSpec((BM, 1), lambda i: (i, 0)),
          pl.BlockSpec((1, HP), lambda i: (0, 0)),
          pl.BlockSpec((BM, 1), lambda i: (i, 0)),
          pl.BlockSpec((HP, FF), lambda i: (0, 0)),
          pl.BlockSpec((1, FF), lambda i: (0, 0)),
          pl.BlockSpec((FF, OUT), lambda i: (0, 0)),
          pl.BlockSpec((1, OUT), lambda i: (0, 0)),
      ],
      out_specs=pl.BlockSpec((B, OUT), lambda i: (0, 0)),
      out_shape=jax.ShapeDtypeStruct((B, OUT), jnp.float32),
      scratch_shapes=[pltpu.VMEM((B, HP), jnp.float32)],
      compiler_params=pltpu.CompilerParams(
          dimension_semantics=("arbitrary",)),
  )(s2, g2, dinv, b2r, batch2d, w3, b3r, w4, b4r)


def kernel(x, edge_index, batch, W1, b1, W2, b2, W3, b3, W4, b4):
  src = edge_index[0]
  dst = edge_index[1]
  pad = EP - E
  src_pad = jnp.concatenate([src, jnp.zeros((pad,), jnp.int32)])
  dst_pad = jnp.concatenate(
      [dst, jnp.full((pad,), EDGE_SENTINEL, jnp.int32)])
  zeros_n1 = jnp.zeros((N + 1,), jnp.float32)
  zrows = jnp.zeros((STRIPE, HP), jnp.float32)
  batch2d = batch.reshape(N, 1)
  hpad = HP - H
  W1p = jnp.pad(W1, ((0, 0), (0, hpad)))
  W2p = jnp.pad(W2, ((0, hpad), (0, hpad)))
  W3p = jnp.pad(W3, ((0, hpad), (0, 0)))
  b1r = jnp.pad(b1, (0, hpad)).reshape(1, HP)
  b2r = jnp.pad(b2, (0, hpad)).reshape(1, HP)
  b3r = b3.reshape(1, FF)
  b4r = b4.reshape(1, OUT)

  degp = _deg_counts(dst_pad, zeros_n1)
  degp_t = degp.T[:N]
  g1, dinv = _g1_kernel(x, W1p, degp_t)
  s1 = _edge_scatter(g1, src_pad, dst_pad, zrows)
  g2 = _g2_kernel(s1[:N], g1, dinv, b1r, W2p)
  s2 = _edge_scatter(g2, src_pad, dst_pad, zrows)
  return _pool_kernel(s2[:N], g2, dinv, b2r, batch2d, W3p, b3r, W4, b4r)


# speedup vs baseline: 1.9149x; 1.9149x over previous
import functools

import jax
import jax.numpy as jnp
from jax import lax
from jax.experimental import pallas as pl
from jax.experimental.pallas import tpu as pltpu
from jax.experimental.pallas import tpu_sc as plsc

N = 50000
E = 800000
B = 512
DF = 78
H = 300
FF = 1024
OUT = 128

NC = 2
NS = 16
LANES = 16

EDGE_SENTINEL = 1 << 29
EP = 819200
DEG_CHUNK = EP // (NC * NS)
SUB = 6400
NSUB_DEG = DEG_CHUNK // SUB
NVREG = SUB // LANES

SCAT_CHUNK = EP // NS
SSUB = 2048
NSUB_SCAT = SCAT_CHUNK // SSUB
SVREG = SSUB // LANES
CHUNK = 32
CSH = 5
KROWS = SSUB // CHUNK

NBLK = 14
NBS = 3840
NP = NBLK * NBS
BLK_PER_SC = NBLK // NC
STRIPE = NBS // NS
HP = 384
SC_TILING_TC = False

BM = 1000
NGRID = N // BM


def _sc_mesh():
  return plsc.VectorSubcoreMesh(
      core_axis_name="c", subcore_axis_name="s", num_cores=NC,
      num_subcores=NS)


def _deg_body(dst_hbm, zeros_hbm, deg_out, acc, dst_sub):
  c = lax.axis_index("c")
  s = lax.axis_index("s")
  wid = c * NS + s
  ones16 = jnp.ones((LANES,), jnp.float32)
  pltpu.sync_copy(zeros_hbm, acc)

  def sub_body(u, _):
    base = wid * DEG_CHUNK + u * SUB
    pltpu.sync_copy(dst_hbm.at[pl.ds(base, SUB)], dst_sub)

    def vec_body(i, _):
      d16 = dst_sub[pl.ds(i * LANES, LANES)]
      idx = jnp.minimum(d16, N)
      plsc.addupdate_scatter(acc, [idx], ones16)
      return 0

    lax.fori_loop(0, NVREG, vec_body, 0)
    return 0

  lax.fori_loop(0, NSUB_DEG, sub_body, 0)
  pltpu.sync_copy(acc, deg_out.at[wid])


def _deg_counts(dst_pad, zeros_n1):
  kfn = pl.kernel(
      _deg_body,
      out_type=jax.ShapeDtypeStruct((NC * NS, N + 1), jnp.float32),
      mesh=_sc_mesh(),
      scratch_types=[
          pltpu.VMEM((N + 1,), jnp.float32),
          pltpu.VMEM((SUB,), jnp.int32),
      ],
      compiler_params=pltpu.CompilerParams(needs_layout_passes=False),
  )
  return kfn(dst_pad, zeros_n1)


def _scatter_body(g_hbm, src_hbm, dst_hbm, zrows_hbm, s_out,
                  acc, src_sub, dst_sub, gsrc_idx, sdst_idx, rowbuf, sem):
  c = lax.axis_index("c")
  s = lax.axis_index("s")

  def blk_body(kb, _):
    blk = c * BLK_PER_SC + kb
    lo = blk * NBS
    hi = lo + NBS

    pltpu.sync_copy(zrows_hbm, acc.at[pl.ds(s * STRIPE, STRIPE)])
    plsc.subcore_barrier()

    def sub_body(u, _):
      base = s * SCAT_CHUNK + u * SSUB
      pltpu.sync_copy(src_hbm.at[pl.ds(base, SSUB)], src_sub)
      pltpu.sync_copy(dst_hbm.at[pl.ds(base, SSUB)], dst_sub)

      def compact(i, n):
        s16 = src_sub[pl.ds(i * LANES, LANES)]
        d16 = dst_sub[pl.ds(i * LANES, LANES)]
        m = (d16 >= lo) & (d16 < hi)
        inc = m.astype(jnp.int32)
        cs = plsc.cumsum(inc)
        pos = (cs - inc) + n
        row = lax.shift_right_logical(pos, CSH)
        col = lax.bitwise_and(pos, CHUNK - 1)
        plsc.store_scatter(gsrc_idx, [row, col], s16, mask=m)
        plsc.store_scatter(sdst_idx, [row, col], d16 - lo, mask=m)
        return n + jnp.sum(inc)

      n = lax.fori_loop(0, SVREG, compact, jnp.int32(0))
      nch = (n + CHUNK - 1) >> CSH

      iota16 = lax.iota(jnp.int32, LANES)
      zero16 = jnp.zeros((LANES,), jnp.int32)
      trash16 = jnp.full((LANES,), NBS, jnp.int32) + s

      def pad_body(t, _):
        p = n + t * LANES + iota16
        m = p < (nch << CSH)
        row = lax.shift_right_logical(p, CSH)
        col = lax.bitwise_and(p, CHUNK - 1)
        plsc.store_scatter(gsrc_idx, [row, col], zero16, mask=m)
        plsc.store_scatter(sdst_idx, [row, col], trash16, mask=m)
        return 0
      lax.fori_loop(0, CHUNK // LANES, pad_body, 0)

      def xfer(j, _):
        pltpu.async_copy(g_hbm.at[gsrc_idx.at[j]], rowbuf, sem).wait()
        pltpu.sync_copy(rowbuf, acc.at[sdst_idx.at[j]], add=True)
        return 0
      lax.fori_loop(0, nch, xfer, 0)
      return 0

    lax.fori_loop(0, NSUB_SCAT, sub_body, 0)
    plsc.subcore_barrier()

    r = s * STRIPE
    pltpu.sync_copy(acc.at[pl.ds(r, STRIPE)], s_out.at[pl.ds(lo + r, STRIPE)])
    plsc.subcore_barrier()
    return 0

  lax.fori_loop(0, BLK_PER_SC, blk_body, 0)


def _edge_scatter(g, src_pad, dst_pad, zrows):
  kfn = pl.kernel(
      _scatter_body,
      out_type=jax.ShapeDtypeStruct((NP, HP), jnp.float32),
      mesh=_sc_mesh(),
      scratch_types=[
          pltpu.VMEM_SHARED((NBS + NS, HP), jnp.float32),
          pltpu.VMEM((SSUB,), jnp.int32),
          pltpu.VMEM((SSUB,), jnp.int32),
          pltpu.VMEM((KROWS, CHUNK), jnp.int32),
          pltpu.VMEM((KROWS, CHUNK), jnp.int32),
          pltpu.VMEM((CHUNK, HP), jnp.float32),
          pltpu.SemaphoreType.DMA,
      ],
      compiler_params=pltpu.CompilerParams(
          needs_layout_passes=False, use_tc_tiling_on_sc=SC_TILING_TC),
  )
  return kfn(g, src_pad, dst_pad, zrows)


def _g1_body(x_ref, w1_ref, degp_ref, g1_ref, dinv_ref):
  deg = degp_ref[...]
  ones = jnp.ones((NC * NS, 1), jnp.float32)
  dsum = jnp.dot(deg, ones, preferred_element_type=jnp.float32)
  dinv = lax.rsqrt(dsum + 1.0)
  h = jnp.dot(x_ref[...], w1_ref[...], preferred_element_type=jnp.float32)
  g1_ref[...] = dinv * h
  dinv_ref[...] = dinv


def _g1_kernel(x, w1, degp):
  return pl.pallas_call(
      _g1_body,
      grid=(NGRID,),
      in_specs=[
          pl.BlockSpec((BM, DF), lambda i: (i, 0)),
          pl.BlockSpec((DF, HP), lambda i: (0, 0)),
          pl.BlockSpec((BM, NC * NS), lambda i: (i, 0)),
      ],
      out_specs=[
          pl.BlockSpec((BM, HP), lambda i: (i, 0)),
          pl.BlockSpec((BM, 1), lambda i: (i, 0)),
      ],
      out_shape=[
          jax.ShapeDtypeStruct((N, HP), jnp.float32),
          jax.ShapeDtypeStruct((N, 1), jnp.float32),
      ],
      compiler_params=pltpu.CompilerParams(
          dimension_semantics=("parallel",)),
  )(x, w1, degp)


def _g2_body(s1_ref, g1_ref, dinv_ref, b1_ref, w2_ref, g2_ref):
  dinv = dinv_ref[...]
  y1 = jnp.maximum(dinv * (s1_ref[...] + g1_ref[...]) + b1_ref[...], 0.0)
  h = jnp.dot(y1, w2_ref[...], preferred_element_type=jnp.float32)
  g2_ref[...] = dinv * h


def _g2_kernel(s1, g1, dinv, b1r, w2):
  return pl.pallas_call(
      _g2_body,
      grid=(NGRID,),
      in_specs=[
          pl.BlockSpec((BM, HP), lambda i: (i, 0)),
          pl.BlockSpec((BM, HP), lambda i: (i, 0)),
          pl.BlockSpec((BM, 1), lambda i: (i, 0)),
          pl.BlockSpec((1, HP), lambda i: (0, 0)),
          pl.BlockSpec((HP, HP), lambda i: (0, 0)),
      ],
      out_specs=pl.BlockSpec((BM, HP), lambda i: (i, 0)),
      out_shape=jax.ShapeDtypeStruct((N, HP), jnp.float32),
      compiler_params=pltpu.CompilerParams(
          dimension_semantics=("parallel",)),
  )(s1, g1, dinv, b1r, w2)


def _pool_body(s2_ref, g2_ref, dinv_ref, b2_ref, batch_ref,
               w3_ref, b3_ref, w4_ref, b4_ref, out_ref, pooled):
  i = pl.program_id(0)

  @pl.when(i == 0)
  def _():
    pooled[...] = jnp.full((B, HP), -1e30, jnp.float32)

  dinv = dinv_ref[...]
  y2 = jnp.maximum(dinv * (s2_ref[...] + g2_ref[...]) + b2_ref[...], 0.0)
  bvec = batch_ref[...]
  g_first = bvec[0, 0]
  g_last = bvec[BM - 1, 0]

  def seg_body(g, _):
    m = bvec == g
    part = jnp.max(jnp.where(m, y2, -1e30), axis=0, keepdims=True)
    cur = pooled[pl.ds(g, 1), :]
    pooled[pl.ds(g, 1), :] = jnp.maximum(cur, part)
    return 0

  lax.fori_loop(g_first, g_last + 1, seg_body, 0)

  @pl.when(i == NGRID - 1)
  def _():
    p = jnp.maximum(pooled[...], 0.0)
    h3 = jnp.maximum(
        jnp.dot(p, w3_ref[...], preferred_element_type=jnp.float32)
        + b3_ref[...], 0.0)
    o = jnp.maximum(
        jnp.dot(h3, w4_ref[...], preferred_element_type=jnp.float32)
        + b4_ref[...], 0.0)
    out_ref[...] = o


def _pool_kernel(s2, g2, dinv, b2r, batch2d, w3, b3r, w4, b4r):
  return pl.pallas_call(
      _pool_body,
      grid=(NGRID,),
      in_specs=[
          pl.BlockSpec((BM, HP), lambda i: (i, 0)),
          pl.BlockSpec((BM, HP), lambda i: (i, 0)),
          pl.BlockSpec((BM, 1), lambda i: (i, 0)),
          pl.BlockSpec((1, HP), lambda i: (0, 0)),
          pl.BlockSpec((BM, 1), lambda i: (i, 0)),
          pl.BlockSpec((HP, FF), lambda i: (0, 0)),
          pl.BlockSpec((1, FF), lambda i: (0, 0)),
          pl.BlockSpec((FF, OUT), lambda i: (0, 0)),
          pl.BlockSpec((1, OUT), lambda i: (0, 0)),
      ],
      out_specs=pl.BlockSpec((B, OUT), lambda i: (0, 0)),
      out_shape=jax.ShapeDtypeStruct((B, OUT), jnp.float32),
      scratch_shapes=[pltpu.VMEM((B, HP), jnp.float32)],
      compiler_params=pltpu.CompilerParams(
          dimension_semantics=("arbitrary",)),
  )(s2, g2, dinv, b2r, batch2d, w3, b3r, w4, b4r)


def kernel(x, edge_index, batch, W1, b1, W2, b2, W3, b3, W4, b4):
  src = edge_index[0]
  dst = edge_index[1]
  pad = EP - E
  src_pad = jnp.concatenate([src, jnp.zeros((pad,), jnp.int32)])
  dst_pad = jnp.concatenate(
      [dst, jnp.full((pad,), EDGE_SENTINEL, jnp.int32)])
  zeros_n1 = jnp.zeros((N + 1,), jnp.float32)
  zrows = jnp.zeros((STRIPE, HP), jnp.float32)
  batch2d = batch.reshape(N, 1)
  hpad = HP - H
  W1p = jnp.pad(W1, ((0, 0), (0, hpad)))
  W2p = jnp.pad(W2, ((0, hpad), (0, hpad)))
  W3p = jnp.pad(W3, ((0, hpad), (0, 0)))
  b1r = jnp.pad(b1, (0, hpad)).reshape(1, HP)
  b2r = jnp.pad(b2, (0, hpad)).reshape(1, HP)
  b3r = b3.reshape(1, FF)
  b4r = b4.reshape(1, OUT)

  degp = _deg_counts(dst_pad, zeros_n1)
  degp_t = degp.T[:N]
  g1, dinv = _g1_kernel(x, W1p, degp_t)
  s1 = _edge_scatter(g1, src_pad, dst_pad, zrows)
  g2 = _g2_kernel(s1[:N], g1, dinv, b1r, W2p)
  s2 = _edge_scatter(g2, src_pad, dst_pad, zrows)
  return _pool_kernel(s2[:N], g2, dinv, b2r, batch2d, W3p, b3r, W4, b4r)


# speedup vs baseline: 1.9512x; 1.0189x over previous
import functools

import jax
import jax.numpy as jnp
from jax import lax
from jax.experimental import pallas as pl
from jax.experimental.pallas import tpu as pltpu
from jax.experimental.pallas import tpu_sc as plsc

N = 50000
E = 800000
B = 512
DF = 78
H = 300
FF = 1024
OUT = 128

NC = 2
NS = 16
LANES = 16

EDGE_SENTINEL = 1 << 29
EP = 819200
DEG_CHUNK = EP // (NC * NS)
SUB = 6400
NSUB_DEG = DEG_CHUNK // SUB
NVREG = SUB // LANES

SCAT_CHUNK = EP // NS
SSUB = 2048
NSUB_SCAT = SCAT_CHUNK // SSUB
SVREG = SSUB // LANES
CHUNK = 32
CSH = 5
KROWS = SSUB // CHUNK

NBLK = 14
NBS = 3840
NP = NBLK * NBS
BLK_PER_SC = NBLK // NC
STRIPE = NBS // NS
HP = 384
SC_TILING_TC = False

BM = 1000
NGRID = N // BM


def _sc_mesh():
  return plsc.VectorSubcoreMesh(
      core_axis_name="c", subcore_axis_name="s", num_cores=NC,
      num_subcores=NS)


def _deg_body(dst_hbm, zeros_hbm, deg_out, acc, dst_sub):
  c = lax.axis_index("c")
  s = lax.axis_index("s")
  wid = c * NS + s
  ones16 = jnp.ones((LANES,), jnp.float32)
  pltpu.sync_copy(zeros_hbm, acc)

  def sub_body(u, _):
    base = wid * DEG_CHUNK + u * SUB
    pltpu.sync_copy(dst_hbm.at[pl.ds(base, SUB)], dst_sub)

    def vec_body(i, _):
      d16 = dst_sub[pl.ds(i * LANES, LANES)]
      idx = jnp.minimum(d16, N)
      plsc.addupdate_scatter(acc, [idx], ones16)
      return 0

    lax.fori_loop(0, NVREG, vec_body, 0)
    return 0

  lax.fori_loop(0, NSUB_DEG, sub_body, 0)
  pltpu.sync_copy(acc, deg_out.at[wid])


def _deg_counts(dst_pad, zeros_n1):
  kfn = pl.kernel(
      _deg_body,
      out_type=jax.ShapeDtypeStruct((NC * NS, N + 1), jnp.float32),
      mesh=_sc_mesh(),
      scratch_types=[
          pltpu.VMEM((N + 1,), jnp.float32),
          pltpu.VMEM((SUB,), jnp.int32),
      ],
      compiler_params=pltpu.CompilerParams(needs_layout_passes=False),
  )
  return kfn(dst_pad, zeros_n1)


def _scatter_body(g_hbm, src_hbm, dst_hbm, zrows_hbm, s_out,
                  acc, src_sub, dst_sub, gsrc_idx, sdst_idx, rb0, rb1,
                  sem0, sem1):
  c = lax.axis_index("c")
  s = lax.axis_index("s")

  def blk_body(kb, _):
    blk = c * BLK_PER_SC + kb
    lo = blk * NBS
    hi = lo + NBS

    pltpu.sync_copy(zrows_hbm, acc.at[pl.ds(s * STRIPE, STRIPE)])
    plsc.subcore_barrier()

    def sub_body(u, _):
      base = s * SCAT_CHUNK + u * SSUB
      pltpu.sync_copy(src_hbm.at[pl.ds(base, SSUB)], src_sub)
      pltpu.sync_copy(dst_hbm.at[pl.ds(base, SSUB)], dst_sub)

      def compact(i, n):
        s16 = src_sub[pl.ds(i * LANES, LANES)]
        d16 = dst_sub[pl.ds(i * LANES, LANES)]
        m = (d16 >= lo) & (d16 < hi)
        inc = m.astype(jnp.int32)
        cs = plsc.cumsum(inc)
        pos = (cs - inc) + n
        row = lax.shift_right_logical(pos, CSH)
        col = lax.bitwise_and(pos, CHUNK - 1)
        plsc.store_scatter(gsrc_idx, [row, col], s16, mask=m)
        plsc.store_scatter(sdst_idx, [row, col], d16 - lo, mask=m)
        return n + jnp.sum(inc)

      n = lax.fori_loop(0, SVREG, compact, jnp.int32(0))
      nch = (n + CHUNK - 1) >> CSH

      iota16 = lax.iota(jnp.int32, LANES)
      zero16 = jnp.zeros((LANES,), jnp.int32)
      trash16 = jnp.full((LANES,), NBS, jnp.int32) + s

      def pad_body(t, _):
        p = n + t * LANES + iota16
        m = p < (nch << CSH)
        row = lax.shift_right_logical(p, CSH)
        col = lax.bitwise_and(p, CHUNK - 1)
        plsc.store_scatter(gsrc_idx, [row, col], zero16, mask=m)
        plsc.store_scatter(sdst_idx, [row, col], trash16, mask=m)
        return 0
      lax.fori_loop(0, CHUNK // LANES, pad_body, 0)

      @pl.when(nch > 0)
      def _():
        pltpu.async_copy(g_hbm.at[gsrc_idx.at[0]], rb0, sem0)

      def pair(q, _):
        j0 = 2 * q
        j1 = j0 + 1

        @pl.when(j1 < nch)
        def _():
          pltpu.async_copy(g_hbm.at[gsrc_idx.at[j1]], rb1, sem1)

        pltpu.make_async_copy(g_hbm.at[gsrc_idx.at[j0]], rb0, sem0).wait()
        pltpu.sync_copy(rb0, acc.at[sdst_idx.at[j0]], add=True)

        @pl.when(j1 < nch)
        def _():
          @pl.when(j1 + 1 < nch)
          def _():
            pltpu.async_copy(g_hbm.at[gsrc_idx.at[j1 + 1]], rb0, sem0)
          pltpu.make_async_copy(g_hbm.at[gsrc_idx.at[j1]], rb1, sem1).wait()
          pltpu.sync_copy(rb1, acc.at[sdst_idx.at[j1]], add=True)
        return 0

      lax.fori_loop(0, (nch + 1) >> 1, pair, 0)
      return 0

    lax.fori_loop(0, NSUB_SCAT, sub_body, 0)
    plsc.subcore_barrier()

    r = s * STRIPE
    pltpu.sync_copy(acc.at[pl.ds(r, STRIPE)], s_out.at[pl.ds(lo + r, STRIPE)])
    plsc.subcore_barrier()
    return 0

  lax.fori_loop(0, BLK_PER_SC, blk_body, 0)


def _edge_scatter(g, src_pad, dst_pad, zrows):
  kfn = pl.kernel(
      _scatter_body,
      out_type=jax.ShapeDtypeStruct((NP, HP), jnp.float32),
      mesh=_sc_mesh(),
      scratch_types=[
          pltpu.VMEM_SHARED((NBS + NS, HP), jnp.float32),
          pltpu.VMEM((SSUB,), jnp.int32),
          pltpu.VMEM((SSUB,), jnp.int32),
          pltpu.VMEM((KROWS, CHUNK), jnp.int32),
          pltpu.VMEM((KROWS, CHUNK), jnp.int32),
          pltpu.VMEM((CHUNK, HP), jnp.float32),
          pltpu.VMEM((CHUNK, HP), jnp.float32),
          pltpu.SemaphoreType.DMA,
          pltpu.SemaphoreType.DMA,
      ],
      compiler_params=pltpu.CompilerParams(
          needs_layout_passes=False, use_tc_tiling_on_sc=SC_TILING_TC),
  )
  return kfn(g, src_pad, dst_pad, zrows)


def _g1_body(x_ref, w1_ref, degp_ref, g1_ref, dinv_ref):
  deg = degp_ref[...]
  ones = jnp.ones((NC * NS, 1), jnp.float32)
  dsum = jnp.dot(deg, ones, preferred_element_type=jnp.float32)
  dinv = lax.rsqrt(dsum + 1.0)
  h = jnp.dot(x_ref[...], w1_ref[...], preferred_element_type=jnp.float32)
  g1_ref[...] = dinv * h
  dinv_ref[...] = dinv


def _g1_kernel(x, w1, degp):
  return pl.pallas_call(
      _g1_body,
      grid=(NGRID,),
      in_specs=[
          pl.BlockSpec((BM, DF), lambda i: (i, 0)),
          pl.BlockSpec((DF, HP), lambda i: (0, 0)),
          pl.BlockSpec((BM, NC * NS), lambda i: (i, 0)),
      ],
      out_specs=[
          pl.BlockSpec((BM, HP), lambda i: (i, 0)),
          pl.BlockSpec((BM, 1), lambda i: (i, 0)),
      ],
      out_shape=[
          jax.ShapeDtypeStruct((N, HP), jnp.float32),
          jax.ShapeDtypeStruct((N, 1), jnp.float32),
      ],
      compiler_params=pltpu.CompilerParams(
          dimension_semantics=("parallel",)),
  )(x, w1, degp)


def _g2_body(s1_ref, g1_ref, dinv_ref, b1_ref, w2_ref, g2_ref):
  dinv = dinv_ref[...]
  y1 = jnp.maximum(dinv * (s1_ref[...] + g1_ref[...]) + b1_ref[...], 0.0)
  h = jnp.dot(y1, w2_ref[...], preferred_element_type=jnp.float32)
  g2_ref[...] = dinv * h


def _g2_kernel(s1, g1, dinv, b1r, w2):
  return pl.pallas_call(
      _g2_body,
      grid=(NGRID,),
      in_specs=[
          pl.BlockSpec((BM, HP), lambda i: (i, 0)),
          pl.BlockSpec((BM, HP), lambda i: (i, 0)),
          pl.BlockSpec((BM, 1), lambda i: (i, 0)),
          pl.BlockSpec((1, HP), lambda i: (0, 0)),
          pl.BlockSpec((HP, HP), lambda i: (0, 0)),
      ],
      out_specs=pl.BlockSpec((BM, HP), lambda i: (i, 0)),
      out_shape=jax.ShapeDtypeStruct((N, HP), jnp.float32),
      compiler_params=pltpu.CompilerParams(
          dimension_semantics=("parallel",)),
  )(s1, g1, dinv, b1r, w2)


def _pool_body(s2_ref, g2_ref, dinv_ref, b2_ref, batch_ref,
               w3_ref, b3_ref, w4_ref, b4_ref, out_ref, pooled):
  i = pl.program_id(0)

  @pl.when(i == 0)
  def _():
    pooled[...] = jnp.full((B, HP), -1e30, jnp.float32)

  dinv = dinv_ref[...]
  y2 = jnp.maximum(dinv * (s2_ref[...] + g2_ref[...]) + b2_ref[...], 0.0)
  bvec = batch_ref[...]
  g_first = bvec[0, 0]
  g_last = bvec[BM - 1, 0]

  def seg_body(g, _):
    m = bvec == g
    part = jnp.max(jnp.where(m, y2, -1e30), axis=0, keepdims=True)
    cur = pooled[pl.ds(g, 1), :]
    pooled[pl.ds(g, 1), :] = jnp.maximum(cur, part)
    return 0

  lax.fori_loop(g_first, g_last + 1, seg_body, 0)

  @pl.when(i == NGRID - 1)
  def _():
    p = jnp.maximum(pooled[...], 0.0)
    h3 = jnp.maximum(
        jnp.dot(p, w3_ref[...], preferred_element_type=jnp.float32)
        + b3_ref[...], 0.0)
    o = jnp.maximum(
        jnp.dot(h3, w4_ref[...], preferred_element_type=jnp.float32)
        + b4_ref[...], 0.0)
    out_ref[...] = o


def _pool_kernel(s2, g2, dinv, b2r, batch2d, w3, b3r, w4, b4r):
  return pl.pallas_call(
      _pool_body,
      grid=(NGRID,),
      in_specs=[
          pl.BlockSpec((BM, HP), lambda i: (i, 0)),
          pl.BlockSpec((BM, HP), lambda i: (i, 0)),
          pl.BlockSpec((BM, 1), lambda i: (i, 0)),
          pl.BlockSpec((1, HP), lambda i: (0, 0)),
          pl.BlockSpec((BM, 1), lambda i: (i, 0)),
          pl.BlockSpec((HP, FF), lambda i: (0, 0)),
          pl.BlockSpec((1, FF), lambda i: (0, 0)),
          pl.BlockSpec((FF, OUT), lambda i: (0, 0)),
          pl.BlockSpec((1, OUT), lambda i: (0, 0)),
      ],
      out_specs=pl.BlockSpec((B, OUT), lambda i: (0, 0)),
      out_shape=jax.ShapeDtypeStruct((B, OUT), jnp.float32),
      scratch_shapes=[pltpu.VMEM((B, HP), jnp.float32)],
      compiler_params=pltpu.CompilerParams(
          dimension_semantics=("arbitrary",)),
  )(s2, g2, dinv, b2r, batch2d, w3, b3r, w4, b4r)


def kernel(x, edge_index, batch, W1, b1, W2, b2, W3, b3, W4, b4):
  src = edge_index[0]
  dst = edge_index[1]
  pad = EP - E
  src_pad = jnp.concatenate([src, jnp.zeros((pad,), jnp.int32)])
  dst_pad = jnp.concatenate(
      [dst, jnp.full((pad,), EDGE_SENTINEL, jnp.int32)])
  zeros_n1 = jnp.zeros((N + 1,), jnp.float32)
  zrows = jnp.zeros((STRIPE, HP), jnp.float32)
  batch2d = batch.reshape(N, 1)
  hpad = HP - H
  W1p = jnp.pad(W1, ((0, 0), (0, hpad)))
  W2p = jnp.pad(W2, ((0, hpad), (0, hpad)))
  W3p = jnp.pad(W3, ((0, hpad), (0, 0)))
  b1r = jnp.pad(b1, (0, hpad)).reshape(1, HP)
  b2r = jnp.pad(b2, (0, hpad)).reshape(1, HP)
  b3r = b3.reshape(1, FF)
  b4r = b4.reshape(1, OUT)

  degp = _deg_counts(dst_pad, zeros_n1)
  degp_t = degp.T[:N]
  g1, dinv = _g1_kernel(x, W1p, degp_t)
  s1 = _edge_scatter(g1, src_pad, dst_pad, zrows)
  g2 = _g2_kernel(s1[:N], g1, dinv, b1r, W2p)
  s2 = _edge_scatter(g2, src_pad, dst_pad, zrows)
  return _pool_kernel(s2[:N], g2, dinv, b2r, batch2d, W3p, b3r, W4, b4r)
